# baseline (speedup 1.0000x reference)
import jax
import jax.numpy as jnp
from jax.experimental import pallas as pl
from jax.experimental.pallas import tpu as pltpu
from jax.experimental.pallas import tpu_sc as plsc

B, S, D = 2, 2048, 2048
E, DE = 8, 1024
TOP_K = 2
DS = 2048
T = B * S
BT = 256
NB = 40
P = NB * BT
RS = 8
QD = D // RS
GW = 128

_f32 = jnp.float32
_bf16 = jnp.bfloat16
_i32 = jnp.int32


def _logits_body(x_ref, wg_ref, o_ref):
    o_ref[...] = jax.lax.dot_general(
        x_ref[...], wg_ref[...], (((1,), (0,)), ((), ())),
        preferred_element_type=_f32)


def _dispatch_body(lg_ref, slots_ref, w2_ref, be_ref, m_scr, s_scr):
    lg = lg_ref[...]
    mx = jnp.max(lg, axis=1, keepdims=True)
    ex = jnp.exp(lg - mx)
    sm = ex / jnp.sum(ex, axis=1, keepdims=True)
    iota_e = jax.lax.broadcasted_iota(_i32, (T, E), 1)
    w1 = jnp.max(sm, axis=1, keepdims=True)
    a1 = jnp.min(jnp.where(sm >= w1, iota_e, E), axis=1, keepdims=True)
    oh1 = iota_e == a1
    smm = jnp.where(oh1, -1e30, sm)
    w2v = jnp.max(smm, axis=1, keepdims=True)
    a2 = jnp.min(jnp.where(smm >= w2v, iota_e, E), axis=1, keepdims=True)
    oh2 = iota_e == a2
    w2_ref[...] = jnp.concatenate([w1, w2v], axis=1)

    m_scr[...] = oh1.astype(_f32) + oh2.astype(_f32)
    CH = 128
    ri = jax.lax.broadcasted_iota(_i32, (CH, CH), 0)
    ci = jax.lax.broadcasted_iota(_i32, (CH, CH), 1)
    tri = (ri > ci).astype(_f32)

    def step(i, carry):
        blk = m_scr[pl.ds(i * CH, CH), :]
        inc = jax.lax.dot_general(tri, blk, (((1,), (0,)), ((), ())),
                                  preferred_element_type=_f32)
        s_scr[pl.ds(i * CH, CH), :] = inc + carry
        return carry + jnp.sum(blk, axis=0, keepdims=True)

    counts = jax.lax.fori_loop(0, T // CH, step, jnp.zeros((1, E), _f32))

    padded = jnp.floor((counts + (BT - 1)) / BT) * BT
    ei = jax.lax.broadcasted_iota(_i32, (E, E), 0)
    ej = jax.lax.broadcasted_iota(_i32, (E, E), 1)
    tri8 = (ei < ej).astype(_f32)
    starts = jax.lax.dot_general(padded, tri8, (((1,), (0,)), ((), ())),
                                 preferred_element_type=_f32)
    ends = starts + padded

    svals = s_scr[...]
    oh1f = oh1.astype(_f32)
    oh2f = oh2.astype(_f32)
    slot1 = jnp.sum(oh1f * (svals + starts), axis=1, keepdims=True)
    slot2 = jnp.sum(oh2f * (svals + starts), axis=1, keepdims=True)
    slots_ref[...] = jnp.concatenate(
        [float(RS) * slot1 + float(r) for r in range(RS)]
        + [float(RS) * slot2 + float(r) for r in range(RS)],
        axis=1).astype(_i32)

    nrow = be_ref.shape[0]
    base = (jax.lax.broadcasted_iota(_i32, (nrow, 1), 0) * BT).astype(_f32)
    cnt = jnp.sum((base >= ends).astype(_i32), axis=1, keepdims=True)
    bexp = jnp.clip(cnt, 0, E - 1)
    nact = (jnp.sum(padded) / BT).astype(_i32)
    rows = jax.lax.broadcasted_iota(_i32, (nrow, 1), 0)
    be_ref[...] = jnp.where(rows < NB, bexp, nact)


def _dot1(a, w_bf16):
    dn = (((1,), (0,)), ((), ()))
    return jax.lax.dot_general(a.astype(_bf16), w_bf16, dn,
                               preferred_element_type=_f32)


def _shared_body(x_ref, wg_ref, wu_ref, wd_ref, o_ref):
    xf = x_ref[...]
    g = _dot1(xf, wg_ref[...])
    u = _dot1(xf, wu_ref[...])
    h = g * jax.nn.sigmoid(g) * u
    o_ref[...] = _dot1(h, wd_ref[...])


def _gemm_body(be_ref, x_ref, wg_ref, wu_ref, wd_ref, y_ref):
    b = pl.program_id(0)
    nact = be_ref[NB]

    @pl.when(b < nact)
    def _():
        xf = x_ref[...]
        g = _dot1(xf, wg_ref[0])
        u = _dot1(xf, wu_ref[0])
        h = g * jax.nn.sigmoid(g) * u
        y_ref[...] = _dot1(h, wd_ref[0])


def _combine_body(sha_ref, shb_ref, ya_ref, yb_ref, w2_ref, o_ref):
    w = w2_ref[...]
    o_ref[...] = (sha_ref[...] + shb_ref[...]
                  + w[:, 0:1] * ya_ref[...].astype(_f32)
                  + w[:, 1:2] * yb_ref[...].astype(_f32))


def _make_sc_calls():
    mesh = plsc.VectorSubcoreMesh(core_axis_name="core",
                                  subcore_axis_name="subcore")

    def scatter_x(xq, sa, sb):
        @pl.kernel(out_type=jax.ShapeDtypeStruct((RS * P, QD), _f32),
                   mesh=mesh)
        def _scatter(xb_hbm, sa_hbm, sb_hbm, xs_hbm):
            def body_a(x_vmem, i_vmem):
                pltpu.sync_copy(x_vmem, xs_hbm.at[i_vmem.at[0]])

            pltpu.emit_pipeline(
                body_a,
                grid=(RS * T // GW,),
                in_specs=[pl.BlockSpec((GW, QD), lambda i: (i, 0)),
                          pl.BlockSpec((1, GW), lambda i: (0, i))],
                out_specs=[],
                core_axis_name=("core", "subcore"),
                dimension_semantics=(pltpu.PARALLEL,),
            )(xb_hbm, sa_hbm)

            def body_b(x_vmem, i_vmem):
                pltpu.sync_copy(x_vmem, xs_hbm.at[i_vmem.at[0]])

            pltpu.emit_pipeline(
                body_b,
                grid=(RS * T // GW,),
                in_specs=[pl.BlockSpec((GW, QD), lambda i: (i, 0)),
                          pl.BlockSpec((1, GW), lambda i: (0, i))],
                out_specs=[],
                core_axis_name=("core", "subcore"),
                dimension_semantics=(pltpu.PARALLEL,),
            )(xb_hbm, sb_hbm)

        return _scatter(xq, sa, sb)

    def gather_y(yq, sa, sb):
        @pl.kernel(out_type=(jax.ShapeDtypeStruct((RS * T, QD), _f32),
                             jax.ShapeDtypeStruct((RS * T, QD), _f32)),
                   mesh=mesh)
        def _gather(y_hbm, sa_hbm, sb_hbm, ya_hbm, yb_hbm):
            def body_a(i_vmem, o_vmem):
                pltpu.sync_copy(y_hbm.at[i_vmem.at[0]], o_vmem)

            pltpu.emit_pipeline(
                body_a,
                grid=(RS * T // GW,),
                in_specs=[pl.BlockSpec((1, GW), lambda i: (0, i))],
                out_specs=[pl.BlockSpec((GW, QD), lambda i: (i, 0))],
                core_axis_name=("core", "subcore"),
                dimension_semantics=(pltpu.PARALLEL,),
            )(sa_hbm, ya_hbm)

            def body_b(i_vmem, o_vmem):
                pltpu.sync_copy(y_hbm.at[i_vmem.at[0]], o_vmem)

            pltpu.emit_pipeline(
                body_b,
                grid=(RS * T // GW,),
                in_specs=[pl.BlockSpec((1, GW), lambda i: (0, i))],
                out_specs=[pl.BlockSpec((GW, QD), lambda i: (i, 0))],
                core_axis_name=("core", "subcore"),
                dimension_semantics=(pltpu.PARALLEL,),
            )(sb_hbm, yb_hbm)

        return _gather(yq, sa, sb)

    return scatter_x, gather_y


def kernel(x, W_g, Wg_e, Wu_e, Wd_e, Ws_gate, Ws_up, Ws_down):
    _scatter_x, _gather_y = _make_sc_calls()
    xf = x.reshape(T, D)
    xb = xf.astype(_bf16)
    wgb = Wg_e.astype(_bf16)
    wub = Wu_e.astype(_bf16)
    wdb = Wd_e.astype(_bf16)
    wsg = Ws_gate.astype(_bf16)
    wsu = Ws_up.astype(_bf16)
    wsd = Ws_down.astype(_bf16)

    logits = pl.pallas_call(
        _logits_body,
        grid=(8,),
        in_specs=[pl.BlockSpec((T // 8, D), lambda i: (i, 0)),
                  pl.BlockSpec((D, E), lambda i: (0, 0))],
        out_specs=pl.BlockSpec((T // 8, E), lambda i: (i, 0)),
        out_shape=jax.ShapeDtypeStruct((T, E), _f32),
    )(xf, W_g)

    slots, w2, be = pl.pallas_call(
        _dispatch_body,
        out_shape=(jax.ShapeDtypeStruct((T, 2 * RS), _i32),
                   jax.ShapeDtypeStruct((T, 2), _f32),
                   jax.ShapeDtypeStruct((48, 1), _i32)),
        scratch_shapes=[pltpu.VMEM((T, E), _f32), pltpu.VMEM((T, E), _f32)],
    )(logits)

    sa = slots[:, 0:RS].reshape(1, RS * T)
    sb = slots[:, RS:2 * RS].reshape(1, RS * T)
    be_flat = be.reshape(48)

    def _shared_half(hi):
        return pl.pallas_call(
            _shared_body,
            grid=(16,),
            in_specs=[pl.BlockSpec((T // 16, D), lambda i: (i, 0)),
                      pl.BlockSpec((D, DS // 2), lambda i, h=hi: (0, h)),
                      pl.BlockSpec((D, DS // 2), lambda i, h=hi: (0, h)),
                      pl.BlockSpec((DS // 2, D), lambda i, h=hi: (h, 0))],
            out_specs=pl.BlockSpec((T // 16, D), lambda i: (i, 0)),
            out_shape=jax.ShapeDtypeStruct((T, D), _f32),
        )(xb, wsg, wsu, wsd)

    shared_a = _shared_half(0)

    x_sorted = _scatter_x(xf.reshape(RS * T, QD), sa, sb).reshape(P, D)

    y = pl.pallas_call(
        _gemm_body,
        grid_spec=pltpu.PrefetchScalarGridSpec(
            num_scalar_prefetch=1,
            grid=(NB,),
            in_specs=[
                pl.BlockSpec((BT, D), lambda b, be_s: (b, 0)),
                pl.BlockSpec((1, D, DE), lambda b, be_s: (be_s[b], 0, 0)),
                pl.BlockSpec((1, D, DE), lambda b, be_s: (be_s[b], 0, 0)),
                pl.BlockSpec((1, DE, D), lambda b, be_s: (be_s[b], 0, 0)),
            ],
            out_specs=pl.BlockSpec((BT, D), lambda b, be_s: (b, 0)),
        ),
        out_shape=jax.ShapeDtypeStruct((P, D), _f32),
        compiler_params=pltpu.CompilerParams(
            dimension_semantics=("arbitrary",)),
    )(be_flat, x_sorted, wgb, wub, wdb)

    shared_b = _shared_half(1)

    yaq, ybq = _gather_y(y.reshape(RS * P, QD), sa, sb)
    ya = yaq.reshape(T, D)
    yb = ybq.reshape(T, D)

    out = pl.pallas_call(
        _combine_body,
        grid=(8,),
        in_specs=[pl.BlockSpec((T // 8, D), lambda i: (i, 0)),
                  pl.BlockSpec((T // 8, D), lambda i: (i, 0)),
                  pl.BlockSpec((T // 8, D), lambda i: (i, 0)),
                  pl.BlockSpec((T // 8, D), lambda i: (i, 0)),
                  pl.BlockSpec((T // 8, 2), lambda i: (i, 0))],
        out_specs=pl.BlockSpec((T // 8, D), lambda i: (i, 0)),
        out_shape=jax.ShapeDtypeStruct((T, D), _f32),
    )(shared_a, shared_b, ya, yb, w2)

    return out.reshape(B, S, D)


# speedup vs baseline: 1.2868x; 1.2868x over previous
import jax
import jax.numpy as jnp
from jax.experimental import pallas as pl
from jax.experimental.pallas import tpu as pltpu
from jax.experimental.pallas import tpu_sc as plsc

B, S, D = 2, 2048, 2048
E, DE = 8, 1024
TOP_K = 2
DS = 2048
T = B * S
BT = 256
NB = 40
P = NB * BT
RS = 8
QD = D // RS
GW = 128

_f32 = jnp.float32
_bf16 = jnp.bfloat16
_i32 = jnp.int32


def _logits_body(x_ref, wg_ref, o_ref):
    o_ref[...] = jax.lax.dot_general(
        x_ref[...], wg_ref[...], (((1,), (0,)), ((), ())),
        preferred_element_type=_f32)


def _dispatch_body(lg_ref, slots_ref, w2_ref, be_ref, m_scr, s_scr):
    lg = lg_ref[...]
    mx = jnp.max(lg, axis=1, keepdims=True)
    ex = jnp.exp(lg - mx)
    sm = ex / jnp.sum(ex, axis=1, keepdims=True)
    iota_e = jax.lax.broadcasted_iota(_i32, (T, E), 1)
    w1 = jnp.max(sm, axis=1, keepdims=True)
    a1 = jnp.min(jnp.where(sm >= w1, iota_e, E), axis=1, keepdims=True)
    oh1 = iota_e == a1
    smm = jnp.where(oh1, -1e30, sm)
    w2v = jnp.max(smm, axis=1, keepdims=True)
    a2 = jnp.min(jnp.where(smm >= w2v, iota_e, E), axis=1, keepdims=True)
    oh2 = iota_e == a2
    w2_ref[...] = jnp.concatenate([w1, w2v], axis=1)

    m_scr[...] = oh1.astype(_f32) + oh2.astype(_f32)
    CH = 128
    ri = jax.lax.broadcasted_iota(_i32, (CH, CH), 0)
    ci = jax.lax.broadcasted_iota(_i32, (CH, CH), 1)
    tri = (ri > ci).astype(_f32)

    def step(i, carry):
        blk = m_scr[pl.ds(i * CH, CH), :]
        inc = jax.lax.dot_general(tri, blk, (((1,), (0,)), ((), ())),
                                  preferred_element_type=_f32)
        s_scr[pl.ds(i * CH, CH), :] = inc + carry
        return carry + jnp.sum(blk, axis=0, keepdims=True)

    counts = jax.lax.fori_loop(0, T // CH, step, jnp.zeros((1, E), _f32))

    padded = jnp.floor((counts + (BT - 1)) / BT) * BT
    ei = jax.lax.broadcasted_iota(_i32, (E, E), 0)
    ej = jax.lax.broadcasted_iota(_i32, (E, E), 1)
    tri8 = (ei < ej).astype(_f32)
    starts = jax.lax.dot_general(padded, tri8, (((1,), (0,)), ((), ())),
                                 preferred_element_type=_f32)
    ends = starts + padded

    svals = s_scr[...]
    oh1f = oh1.astype(_f32)
    oh2f = oh2.astype(_f32)
    slot1 = jnp.sum(oh1f * (svals + starts), axis=1, keepdims=True)
    slot2 = jnp.sum(oh2f * (svals + starts), axis=1, keepdims=True)
    slots_ref[...] = jnp.concatenate(
        [float(RS) * slot1 + float(r) for r in range(RS)]
        + [float(RS) * slot2 + float(r) for r in range(RS)],
        axis=1).astype(_i32)

    nrow = be_ref.shape[0]
    base = (jax.lax.broadcasted_iota(_i32, (nrow, 1), 0) * BT).astype(_f32)
    cnt = jnp.sum((base >= ends).astype(_i32), axis=1, keepdims=True)
    bexp = jnp.clip(cnt, 0, E - 1)
    nact = (jnp.sum(padded) / BT).astype(_i32)
    rows = jax.lax.broadcasted_iota(_i32, (nrow, 1), 0)
    be_ref[...] = jnp.where(rows < NB, bexp, nact)


def _dot1(a, w_bf16):
    dn = (((1,), (0,)), ((), ()))
    return jax.lax.dot_general(a.astype(_bf16), w_bf16, dn,
                               preferred_element_type=_f32)


def _shared_body(x_ref, wg_ref, wu_ref, wd_ref, o_ref):
    xf = x_ref[...]
    g = _dot1(xf, wg_ref[...])
    u = _dot1(xf, wu_ref[...])
    h = g * jax.nn.sigmoid(g) * u
    o_ref[...] = _dot1(h, wd_ref[...])


def _gemm_body(be_ref, x_ref, wg_ref, wu_ref, wd_ref, y_ref):
    b = pl.program_id(0)
    nact = be_ref[NB]

    @pl.when(b < nact)
    def _():
        xs = x_ref[...]
        g = jnp.zeros((BT, DE), _f32)
        u = jnp.zeros((BT, DE), _f32)
        for k in range(RS):
            xk = xs[:, k, :]
            g = g + _dot1(xk, wg_ref[0, k].astype(_bf16))
            u = u + _dot1(xk, wu_ref[0, k].astype(_bf16))
        h = (g * jax.nn.sigmoid(g) * u).astype(_bf16)
        for j in range(RS):
            y_ref[:, j, :] = jax.lax.dot_general(
                h, wd_ref[0, :, :].astype(_bf16)[:, j * QD:(j + 1) * QD],
                (((1,), (0,)), ((), ())), preferred_element_type=_f32)


def _combine_body(sha_ref, shb_ref, ya_ref, yb_ref, w2_ref, o_ref):
    w = w2_ref[...]
    o_ref[...] = (sha_ref[...] + shb_ref[...]
                  + w[:, 0:1] * ya_ref[...].astype(_f32)
                  + w[:, 1:2] * yb_ref[...].astype(_f32))


def _make_sc_calls():
    mesh = plsc.VectorSubcoreMesh(core_axis_name="core",
                                  subcore_axis_name="subcore")

    def scatter_x(xq, sa, sb):
        @pl.kernel(out_type=jax.ShapeDtypeStruct((RS * P, QD), _f32),
                   mesh=mesh)
        def _scatter(xb_hbm, sa_hbm, sb_hbm, xs_hbm):
            def body_a(x_vmem, i_vmem):
                pltpu.sync_copy(x_vmem, xs_hbm.at[i_vmem.at[0]])

            pltpu.emit_pipeline(
                body_a,
                grid=(RS * T // GW,),
                in_specs=[pl.BlockSpec((GW, QD), lambda i: (i, 0)),
                          pl.BlockSpec((1, GW), lambda i: (0, i))],
                out_specs=[],
                core_axis_name=("core", "subcore"),
                dimension_semantics=(pltpu.PARALLEL,),
            )(xb_hbm, sa_hbm)

            def body_b(x_vmem, i_vmem):
                pltpu.sync_copy(x_vmem, xs_hbm.at[i_vmem.at[0]])

            pltpu.emit_pipeline(
                body_b,
                grid=(RS * T // GW,),
                in_specs=[pl.BlockSpec((GW, QD), lambda i: (i, 0)),
                          pl.BlockSpec((1, GW), lambda i: (0, i))],
                out_specs=[],
                core_axis_name=("core", "subcore"),
                dimension_semantics=(pltpu.PARALLEL,),
            )(xb_hbm, sb_hbm)

        return _scatter(xq, sa, sb)

    def gather_y(yq, sa, sb):
        @pl.kernel(out_type=(jax.ShapeDtypeStruct((RS * T, QD), _f32),
                             jax.ShapeDtypeStruct((RS * T, QD), _f32)),
                   mesh=mesh)
        def _gather(y_hbm, sa_hbm, sb_hbm, ya_hbm, yb_hbm):
            def body_a(i_vmem, o_vmem):
                pltpu.sync_copy(y_hbm.at[i_vmem.at[0]], o_vmem)

            pltpu.emit_pipeline(
                body_a,
                grid=(RS * T // GW,),
                in_specs=[pl.BlockSpec((1, GW), lambda i: (0, i))],
                out_specs=[pl.BlockSpec((GW, QD), lambda i: (i, 0))],
                core_axis_name=("core", "subcore"),
                dimension_semantics=(pltpu.PARALLEL,),
            )(sa_hbm, ya_hbm)

            def body_b(i_vmem, o_vmem):
                pltpu.sync_copy(y_hbm.at[i_vmem.at[0]], o_vmem)

            pltpu.emit_pipeline(
                body_b,
                grid=(RS * T // GW,),
                in_specs=[pl.BlockSpec((1, GW), lambda i: (0, i))],
                out_specs=[pl.BlockSpec((GW, QD), lambda i: (i, 0))],
                core_axis_name=("core", "subcore"),
                dimension_semantics=(pltpu.PARALLEL,),
            )(sb_hbm, yb_hbm)

        return _gather(yq, sa, sb)

    return scatter_x, gather_y


def kernel(x, W_g, Wg_e, Wu_e, Wd_e, Ws_gate, Ws_up, Ws_down):
    _scatter_x, _gather_y = _make_sc_calls()
    xf = x.reshape(T, D)
    xb = xf.astype(_bf16)
    wgb = Wg_e.astype(_bf16)
    wub = Wu_e.astype(_bf16)
    wdb = Wd_e.astype(_bf16)
    wsg = Ws_gate.astype(_bf16)
    wsu = Ws_up.astype(_bf16)
    wsd = Ws_down.astype(_bf16)

    logits = pl.pallas_call(
        _logits_body,
        grid=(8,),
        in_specs=[pl.BlockSpec((T // 8, D), lambda i: (i, 0)),
                  pl.BlockSpec((D, E), lambda i: (0, 0))],
        out_specs=pl.BlockSpec((T // 8, E), lambda i: (i, 0)),
        out_shape=jax.ShapeDtypeStruct((T, E), _f32),
    )(xf, W_g)

    slots, w2, be = pl.pallas_call(
        _dispatch_body,
        out_shape=(jax.ShapeDtypeStruct((T, 2 * RS), _i32),
                   jax.ShapeDtypeStruct((T, 2), _f32),
                   jax.ShapeDtypeStruct((48, 1), _i32)),
        scratch_shapes=[pltpu.VMEM((T, E), _f32), pltpu.VMEM((T, E), _f32)],
    )(logits)

    sa = slots[:, 0:RS].reshape(1, RS * T)
    sb = slots[:, RS:2 * RS].reshape(1, RS * T)
    be_flat = be.reshape(48)

    def _shared_half(hi):
        return pl.pallas_call(
            _shared_body,
            grid=(16,),
            in_specs=[pl.BlockSpec((T // 16, D), lambda i: (i, 0)),
                      pl.BlockSpec((D, DS // 2), lambda i, h=hi: (0, h)),
                      pl.BlockSpec((D, DS // 2), lambda i, h=hi: (0, h)),
                      pl.BlockSpec((DS // 2, D), lambda i, h=hi: (h, 0))],
            out_specs=pl.BlockSpec((T // 16, D), lambda i: (i, 0)),
            out_shape=jax.ShapeDtypeStruct((T, D), _f32),
        )(xb, wsg, wsu, wsd)

    shared_a = _shared_half(0)

    x_sorted3 = _scatter_x(xf.reshape(RS * T, QD), sa, sb).reshape(P, RS, QD)

    y3 = pl.pallas_call(
        _gemm_body,
        grid_spec=pltpu.PrefetchScalarGridSpec(
            num_scalar_prefetch=1,
            grid=(NB,),
            in_specs=[
                pl.BlockSpec((BT, RS, QD), lambda b, be_s: (b, 0, 0)),
                pl.BlockSpec((1, RS, QD, DE),
                             lambda b, be_s: (be_s[b], 0, 0, 0)),
                pl.BlockSpec((1, RS, QD, DE),
                             lambda b, be_s: (be_s[b], 0, 0, 0)),
                pl.BlockSpec((1, DE, D), lambda b, be_s: (be_s[b], 0, 0)),
            ],
            out_specs=pl.BlockSpec((BT, RS, QD), lambda b, be_s: (b, 0, 0)),
        ),
        out_shape=jax.ShapeDtypeStruct((P, RS, QD), _f32),
        compiler_params=pltpu.CompilerParams(
            dimension_semantics=("arbitrary",)),
    )(be_flat, x_sorted3, Wg_e.reshape(E, RS, QD, DE),
      Wu_e.reshape(E, RS, QD, DE), Wd_e)

    shared_b = _shared_half(1)

    yaq, ybq = _gather_y(y3.reshape(RS * P, QD), sa, sb)
    ya = yaq.reshape(T, D)
    yb = ybq.reshape(T, D)

    out = pl.pallas_call(
        _combine_body,
        grid=(8,),
        in_specs=[pl.BlockSpec((T // 8, D), lambda i: (i, 0)),
                  pl.BlockSpec((T // 8, D), lambda i: (i, 0)),
                  pl.BlockSpec((T // 8, D), lambda i: (i, 0)),
                  pl.BlockSpec((T // 8, D), lambda i: (i, 0)),
                  pl.BlockSpec((T // 8, 2), lambda i: (i, 0))],
        out_specs=pl.BlockSpec((T // 8, D), lambda i: (i, 0)),
        out_shape=jax.ShapeDtypeStruct((T, D), _f32),
    )(shared_a, shared_b, ya, yb, w2)

    return out.reshape(B, S, D)


# speedup vs baseline: 1.4484x; 1.1256x over previous
import jax
import jax.numpy as jnp
from jax.experimental import pallas as pl
from jax.experimental.pallas import tpu as pltpu
from jax.experimental.pallas import tpu_sc as plsc

B, S, D = 2, 2048, 2048
E, DE = 8, 1024
TOP_K = 2
DS = 2048
T = B * S
BT = 256
NB = 40
P = NB * BT
RS = 8
QD = D // RS
GW = 128

_f32 = jnp.float32
_bf16 = jnp.bfloat16
_i32 = jnp.int32


def _logits_body(x_ref, wg_ref, o_ref, xq_ref, xb_ref):
    xv = x_ref[...]
    o_ref[...] = jax.lax.dot_general(
        xv, wg_ref[...], (((1,), (0,)), ((), ())),
        preferred_element_type=_f32)
    xb_ref[...] = xv.astype(_bf16)
    for j in range(RS):
        xq_ref[:, j, :] = xv[:, j * QD:(j + 1) * QD]


def _dispatch_body(lg_ref, slots_ref, w2_ref, be_ref, m_scr, s_scr):
    lg = lg_ref[...]
    mx = jnp.max(lg, axis=1, keepdims=True)
    ex = jnp.exp(lg - mx)
    sm = ex / jnp.sum(ex, axis=1, keepdims=True)
    iota_e = jax.lax.broadcasted_iota(_i32, (T, E), 1)
    w1 = jnp.max(sm, axis=1, keepdims=True)
    a1 = jnp.min(jnp.where(sm >= w1, iota_e, E), axis=1, keepdims=True)
    oh1 = iota_e == a1
    smm = jnp.where(oh1, -1e30, sm)
    w2v = jnp.max(smm, axis=1, keepdims=True)
    a2 = jnp.min(jnp.where(smm >= w2v, iota_e, E), axis=1, keepdims=True)
    oh2 = iota_e == a2
    w2_ref[...] = jnp.concatenate([w1, w2v], axis=1)

    m_scr[...] = oh1.astype(_f32) + oh2.astype(_f32)
    CH = 128
    ri = jax.lax.broadcasted_iota(_i32, (CH, CH), 0)
    ci = jax.lax.broadcasted_iota(_i32, (CH, CH), 1)
    tri = (ri > ci).astype(_f32)

    def step(i, carry):
        blk = m_scr[pl.ds(i * CH, CH), :]
        inc = jax.lax.dot_general(tri, blk, (((1,), (0,)), ((), ())),
                                  preferred_element_type=_f32)
        s_scr[pl.ds(i * CH, CH), :] = inc + carry
        return carry + jnp.sum(blk, axis=0, keepdims=True)

    counts = jax.lax.fori_loop(0, T // CH, step, jnp.zeros((1, E), _f32))

    padded = jnp.floor((counts + (BT - 1)) / BT) * BT
    ei = jax.lax.broadcasted_iota(_i32, (E, E), 0)
    ej = jax.lax.broadcasted_iota(_i32, (E, E), 1)
    tri8 = (ei < ej).astype(_f32)
    starts = jax.lax.dot_general(padded, tri8, (((1,), (0,)), ((), ())),
                                 preferred_element_type=_f32)
    ends = starts + padded

    svals = s_scr[...]
    oh1f = oh1.astype(_f32)
    oh2f = oh2.astype(_f32)
    slot1 = jnp.sum(oh1f * (svals + starts), axis=1, keepdims=True)
    slot2 = jnp.sum(oh2f * (svals + starts), axis=1, keepdims=True)
    slots_ref[...] = jnp.concatenate(
        [float(RS) * slot1 + float(r) for r in range(RS)]
        + [float(RS) * slot2 + float(r) for r in range(RS)],
        axis=1).astype(_i32)

    nrow = be_ref.shape[0]
    base = (jax.lax.broadcasted_iota(_i32, (nrow, 1), 0) * BT).astype(_f32)
    cnt = jnp.sum((base >= ends).astype(_i32), axis=1, keepdims=True)
    bexp = jnp.clip(cnt, 0, E - 1)
    nact = (jnp.sum(padded) / BT).astype(_i32)
    rows = jax.lax.broadcasted_iota(_i32, (nrow, 1), 0)
    be_ref[...] = jnp.where(rows < NB, bexp, nact)


def _dot1(a, w_bf16):
    dn = (((1,), (0,)), ((), ()))
    return jax.lax.dot_general(a.astype(_bf16), w_bf16, dn,
                               preferred_element_type=_f32)


def _shared_body(x_ref, wg_ref, wu_ref, wd_ref, o_ref):
    xv = x_ref[...]
    g = _dot1(xv, wg_ref[...])
    u = _dot1(xv, wu_ref[...])
    h = (g * jax.nn.sigmoid(g) * u).astype(_bf16)
    for j in range(RS):
        o_ref[:, j, :] = jax.lax.dot_general(
            h, wd_ref[:, pl.ds(j * QD, QD)], (((1,), (0,)), ((), ())),
            preferred_element_type=_f32)


def _gemm_body(be_ref, x_ref, wg_ref, wu_ref, wd_ref, y_ref):
    b = pl.program_id(0)
    nact = be_ref[NB]

    @pl.when(b < nact)
    def _():
        xs = x_ref[...]
        g = jnp.zeros((BT, DE), _f32)
        u = jnp.zeros((BT, DE), _f32)
        for k in range(RS):
            xk = xs[:, k, :]
            g = g + _dot1(xk, wg_ref[0, k].astype(_bf16))
            u = u + _dot1(xk, wu_ref[0, k].astype(_bf16))
        h = (g * jax.nn.sigmoid(g) * u).astype(_bf16)
        for j in range(RS):
            y_ref[:, j, :] = jax.lax.dot_general(
                h, wd_ref[0, :, :].astype(_bf16)[:, j * QD:(j + 1) * QD],
                (((1,), (0,)), ((), ())), preferred_element_type=_f32)


def _combine_body(sh_ref, ya_ref, yb_ref, w2_ref, o_ref):
    w = w2_ref[...]
    wa = w[:, 0:1][:, :, None]
    wb = w[:, 1:2][:, :, None]
    o_ref[...] = sh_ref[...] + wa * ya_ref[...] + wb * yb_ref[...]


def _make_sc_calls():
    mesh = plsc.VectorSubcoreMesh(core_axis_name="core",
                                  subcore_axis_name="subcore")

    def scatter_x(xq, sa, sb):
        @pl.kernel(out_type=jax.ShapeDtypeStruct((RS * P, QD), _f32),
                   mesh=mesh)
        def _scatter(xb_hbm, sa_hbm, sb_hbm, xs_hbm):
            def body_a(x_vmem, i_vmem):
                pltpu.sync_copy(x_vmem, xs_hbm.at[i_vmem.at[0]])

            pltpu.emit_pipeline(
                body_a,
                grid=(RS * T // GW,),
                in_specs=[pl.BlockSpec((GW, QD), lambda i: (i, 0)),
                          pl.BlockSpec((1, GW), lambda i: (0, i))],
                out_specs=[],
                core_axis_name=("core", "subcore"),
                dimension_semantics=(pltpu.PARALLEL,),
            )(xb_hbm, sa_hbm)

            def body_b(x_vmem, i_vmem):
                pltpu.sync_copy(x_vmem, xs_hbm.at[i_vmem.at[0]])

            pltpu.emit_pipeline(
                body_b,
                grid=(RS * T // GW,),
                in_specs=[pl.BlockSpec((GW, QD), lambda i: (i, 0)),
                          pl.BlockSpec((1, GW), lambda i: (0, i))],
                out_specs=[],
                core_axis_name=("core", "subcore"),
                dimension_semantics=(pltpu.PARALLEL,),
            )(xb_hbm, sb_hbm)

        return _scatter(xq, sa, sb)

    def gather_y(yq, sa, sb):
        @pl.kernel(out_type=(jax.ShapeDtypeStruct((RS * T, QD), _f32),
                             jax.ShapeDtypeStruct((RS * T, QD), _f32)),
                   mesh=mesh)
        def _gather(y_hbm, sa_hbm, sb_hbm, ya_hbm, yb_hbm):
            def body_a(i_vmem, o_vmem):
                pltpu.sync_copy(y_hbm.at[i_vmem.at[0]], o_vmem)

            pltpu.emit_pipeline(
                body_a,
                grid=(RS * T // GW,),
                in_specs=[pl.BlockSpec((1, GW), lambda i: (0, i))],
                out_specs=[pl.BlockSpec((GW, QD), lambda i: (i, 0))],
                core_axis_name=("core", "subcore"),
                dimension_semantics=(pltpu.PARALLEL,),
            )(sa_hbm, ya_hbm)

            def body_b(i_vmem, o_vmem):
                pltpu.sync_copy(y_hbm.at[i_vmem.at[0]], o_vmem)

            pltpu.emit_pipeline(
                body_b,
                grid=(RS * T // GW,),
                in_specs=[pl.BlockSpec((1, GW), lambda i: (0, i))],
                out_specs=[pl.BlockSpec((GW, QD), lambda i: (i, 0))],
                core_axis_name=("core", "subcore"),
                dimension_semantics=(pltpu.PARALLEL,),
            )(sb_hbm, yb_hbm)

        return _gather(yq, sa, sb)

    return scatter_x, gather_y


def kernel(x, W_g, Wg_e, Wu_e, Wd_e, Ws_gate, Ws_up, Ws_down):
    _scatter_x, _gather_y = _make_sc_calls()
    xf = x.reshape(T, D)
    wsg = Ws_gate.astype(_bf16)
    wsu = Ws_up.astype(_bf16)
    wsd = Ws_down.astype(_bf16)

    logits, xq, xb = pl.pallas_call(
        _logits_body,
        grid=(8,),
        in_specs=[pl.BlockSpec((T // 8, D), lambda i: (i, 0)),
                  pl.BlockSpec((D, E), lambda i: (0, 0))],
        out_specs=(pl.BlockSpec((T // 8, E), lambda i: (i, 0)),
                   pl.BlockSpec((T // 8, RS, QD), lambda i: (i, 0, 0)),
                   pl.BlockSpec((T // 8, D), lambda i: (i, 0))),
        out_shape=(jax.ShapeDtypeStruct((T, E), _f32),
                   jax.ShapeDtypeStruct((T, RS, QD), _f32),
                   jax.ShapeDtypeStruct((T, D), _bf16)),
    )(xf, W_g)

    slots, w2, be = pl.pallas_call(
        _dispatch_body,
        out_shape=(jax.ShapeDtypeStruct((T, 2 * RS), _i32),
                   jax.ShapeDtypeStruct((T, 2), _f32),
                   jax.ShapeDtypeStruct((48, 1), _i32)),
        scratch_shapes=[pltpu.VMEM((T, E), _f32), pltpu.VMEM((T, E), _f32)],
    )(logits)

    sa = slots[:, 0:RS].reshape(1, RS * T)
    sb = slots[:, RS:2 * RS].reshape(1, RS * T)
    be_flat = be.reshape(48)

    shared = pl.pallas_call(
        _shared_body,
        grid=(16,),
        in_specs=[pl.BlockSpec((T // 16, D), lambda i: (i, 0)),
                  pl.BlockSpec((D, DS), lambda i: (0, 0)),
                  pl.BlockSpec((D, DS), lambda i: (0, 0)),
                  pl.BlockSpec((DS, D), lambda i: (0, 0))],
        out_specs=pl.BlockSpec((T // 16, RS, QD), lambda i: (i, 0, 0)),
        out_shape=jax.ShapeDtypeStruct((T, RS, QD), _f32),
    )(xb, wsg, wsu, wsd)

    x_sorted3 = _scatter_x(xq.reshape(RS * T, QD), sa, sb).reshape(P, RS, QD)

    y3 = pl.pallas_call(
        _gemm_body,
        grid_spec=pltpu.PrefetchScalarGridSpec(
            num_scalar_prefetch=1,
            grid=(NB,),
            in_specs=[
                pl.BlockSpec((BT, RS, QD), lambda b, be_s: (b, 0, 0)),
                pl.BlockSpec((1, RS, QD, DE),
                             lambda b, be_s: (be_s[b], 0, 0, 0)),
                pl.BlockSpec((1, RS, QD, DE),
                             lambda b, be_s: (be_s[b], 0, 0, 0)),
                pl.BlockSpec((1, DE, D), lambda b, be_s: (be_s[b], 0, 0)),
            ],
            out_specs=pl.BlockSpec((BT, RS, QD), lambda b, be_s: (b, 0, 0)),
        ),
        out_shape=jax.ShapeDtypeStruct((P, RS, QD), _f32),
        compiler_params=pltpu.CompilerParams(
            dimension_semantics=("arbitrary",)),
    )(be_flat, x_sorted3, Wg_e.reshape(E, RS, QD, DE),
      Wu_e.reshape(E, RS, QD, DE), Wd_e)

    yaq, ybq = _gather_y(y3.reshape(RS * P, QD), sa, sb)
    ya3 = yaq.reshape(T, RS, QD)
    yb3 = ybq.reshape(T, RS, QD)

    out3 = pl.pallas_call(
        _combine_body,
        grid=(8,),
        in_specs=[pl.BlockSpec((T // 8, RS, QD), lambda i: (i, 0, 0)),
                  pl.BlockSpec((T // 8, RS, QD), lambda i: (i, 0, 0)),
                  pl.BlockSpec((T // 8, RS, QD), lambda i: (i, 0, 0)),
                  pl.BlockSpec((T // 8, 2), lambda i: (i, 0))],
        out_specs=pl.BlockSpec((T // 8, RS, QD), lambda i: (i, 0, 0)),
        out_shape=jax.ShapeDtypeStruct((T, RS, QD), _f32),
    )(shared, ya3, yb3, w2)

    return out3.reshape(B, S, D)
